# baseline (speedup 1.0000x reference)
import functools

import jax
import jax.numpy as jnp
from jax import lax
from jax.experimental import pallas as pl
from jax.experimental.pallas import tpu as pltpu
from jax.experimental.pallas import tpu_sc as plsc

N = 10000
E = 320000
NC = 2
NS = 16
LANES = 16
NW = NC * NS
EPW = E // NW
CH = 80
NCHUNK = EPW // CH
NP = 10240
RPT = NP // NS
ZR = 8


def _make_sc_edge_pass(H, F):
    HF = H * F
    ROW = HF + 8
    mesh = plsc.VectorSubcoreMesh(core_axis_name="c", subcore_axis_name="s")

    def _pair(ty):
        return [ty, ty]

    @functools.partial(
        pl.kernel,
        out_type=jax.ShapeDtypeStruct((NC, NP, ROW), jnp.float32),
        mesh=mesh,
        compiler_params=pltpu.CompilerParams(
            needs_layout_passes=False, use_tc_tiling_on_sc=False),
        scratch_types=[
            pltpu.VMEM((CH,), jnp.int32),
            _pair(pltpu.VMEM((CH,), jnp.int32)),
            pltpu.VMEM((CH, 16), jnp.float32),
            pltpu.VMEM((CH, 16), jnp.float32),
            pltpu.VMEM((CH, HF), jnp.float32),
            _pair(pltpu.VMEM((CH, ROW), jnp.float32)),
            pltpu.VMEM((CH * LANES,), jnp.float32),
            pltpu.VMEM_SHARED((NP, ROW), jnp.float32),
            pltpu.SemaphoreType.DMA,
            _pair(pltpu.SemaphoreType.DMA),
        ],
    )
    def sc_pass(src_hbm, dst_hbm, a_hbm, a2_hbm, h_hbm, out_hbm,
                srcv, dstv, asrcv, adstv, hv, rowv, wv, accum,
                sem_g, sem_sc):
        cid = lax.axis_index("c")
        sid = lax.axis_index("s")
        wid = cid * NS + sid
        zvec = jnp.zeros((LANES,), jnp.float32)
        lane = lax.iota(jnp.int32, LANES)
        wmask = lane < H

        @pl.loop(0, CH)
        def _(r):
            for k in range(ROW // LANES):
                rowv[0][r, pl.ds(k * LANES, LANES)] = zvec

        @pl.loop(0, RPT // CH)
        def _(z):
            pltpu.sync_copy(rowv[0], accum.at[pl.ds(sid * RPT + z * CH, CH)])

        plsc.subcore_barrier()

        def wait_scatter(p):
            pltpu.make_async_copy(
                rowv[p], accum.at[dstv[p]], sem_sc[p]).wait()

        def chunk_body(p, ci, guarded):
            if guarded:
                @pl.when(ci >= 2)
                def _():
                    wait_scatter(p)
            else:
                wait_scatter(p)
            off = wid * EPW + ci * CH
            pltpu.sync_copy(src_hbm.at[pl.ds(off, CH)], srcv)
            pltpu.sync_copy(dst_hbm.at[pl.ds(off, CH)], dstv[p])
            c1 = pltpu.async_copy(a_hbm.at[srcv], asrcv, sem_g)
            c2 = pltpu.async_copy(a2_hbm.at[dstv[p]], adstv, sem_g)
            c3 = pltpu.async_copy(h_hbm.at[srcv], hv, sem_g)
            c1.wait()
            c2.wait()
            c3.wait()

            @pl.loop(0, CH, unroll=2)
            def _(e):
                s = asrcv[e, :] + adstv[e, :]
                s = jnp.where(s >= 0.0, s, 0.2 * s)
                w = jnp.where(wmask, jnp.exp(s), 0.0)
                rowv[p][e, pl.ds(0, LANES)] = w
                wv[pl.ds(e * LANES, LANES)] = w
                for hd in range(H):
                    whd = plsc.load_gather(
                        wv, [jnp.full((LANES,), e * LANES + hd, jnp.int32)])
                    for fv in range(F // LANES):
                        col = hd * F + fv * LANES
                        rowv[p][e, pl.ds(8 + col, LANES)] = (
                            hv[e, pl.ds(col, LANES)] * whd)

            pltpu.async_copy(rowv[p], accum.at[dstv[p]], sem_sc[p], add=True)

        @pl.loop(0, NCHUNK - 1, step=2)
        def _(ci):
            chunk_body(0, ci, True)
            chunk_body(1, ci + 1, True)

        chunk_body(0, NCHUNK - 1, False)
        wait_scatter(1)
        wait_scatter(0)

        plsc.subcore_barrier()
        pltpu.sync_copy(accum.at[pl.ds(sid * RPT, RPT)],
                        out_hbm.at[cid, pl.ds(sid * RPT, RPT)])

    return sc_pass


_sc_pass_l1 = _make_sc_edge_pass(8, 16)
_sc_pass_l2 = _make_sc_edge_pass(1, 64)


def _mm_body(x_ref, w_ref, o_ref):
    o_ref[...] = jnp.dot(x_ref[...], w_ref[...],
                         preferred_element_type=jnp.float32)


def _tc_matmul(x, w, block_rows=2000):
    n, k = x.shape
    _, m = w.shape
    return pl.pallas_call(
        _mm_body,
        grid=(n // block_rows,),
        in_specs=[
            pl.BlockSpec((block_rows, k), lambda i: (i, 0)),
            pl.BlockSpec((k, m), lambda i: (0, 0)),
        ],
        out_specs=pl.BlockSpec((block_rows, m), lambda i: (i, 0)),
        out_shape=jax.ShapeDtypeStruct((n, m), jnp.float32),
    )(x, w)


def _ep1_body(acc_ref, b_ref, rep_ref, w2_ref, o_ref):
    a = acc_ref[0] + acc_ref[1]
    msg = a[:, 8:136]
    den = jnp.dot(a[:, 0:8], rep_ref[...],
                  preferred_element_type=jnp.float32)
    out1 = msg / (den + 1e-16) + b_ref[...]
    x2 = jnp.where(out1 > 0.0, out1, jnp.exp(jnp.minimum(out1, 0.0)) - 1.0)
    o_ref[...] = jnp.dot(x2, w2_ref[...], preferred_element_type=jnp.float32)


def _tc_epilogue1(acc, b1, rep, Wcat2, block_rows=2000):
    return pl.pallas_call(
        _ep1_body,
        grid=(N // block_rows,),
        in_specs=[
            pl.BlockSpec((2, block_rows, 136), lambda i: (0, i, 0)),
            pl.BlockSpec((1, 128), lambda i: (0, 0)),
            pl.BlockSpec((8, 128), lambda i: (0, 0)),
            pl.BlockSpec((128, 128), lambda i: (0, 0)),
        ],
        out_specs=pl.BlockSpec((block_rows, 128), lambda i: (i, 0)),
        out_shape=jax.ShapeDtypeStruct((N, 128), jnp.float32),
    )(acc, b1, rep, Wcat2)


def _ep2_body(acc_ref, b_ref, o_ref):
    a = acc_ref[0] + acc_ref[1]
    den = a[:, 0:1]
    o_ref[...] = a[:, 8:72] / (den + 1e-16) + b_ref[...]


def _tc_epilogue2(acc, b2, block_rows=2000):
    return pl.pallas_call(
        _ep2_body,
        grid=(N // block_rows,),
        in_specs=[
            pl.BlockSpec((2, block_rows, 72), lambda i: (0, i, 0)),
            pl.BlockSpec((1, 64), lambda i: (0, 0)),
        ],
        out_specs=pl.BlockSpec((block_rows, 64), lambda i: (i, 0)),
        out_shape=jax.ShapeDtypeStruct((N, 64), jnp.float32),
    )(acc, b2)


def kernel(x, edge_index, W1, att_src1, att_dst1, b1, W2, att_src2, att_dst2, b2):
    src = edge_index[0].astype(jnp.int32)
    dst = edge_index[1].astype(jnp.int32)

    H1, F1 = att_src1.shape
    H2, F2 = att_src2.shape

    Wsrc1 = (W1.reshape(128, H1, F1) * att_src1[None]).sum(-1)
    Wdst1 = (W1.reshape(128, H1, F1) * att_dst1[None]).sum(-1)
    Wcat1 = jnp.concatenate([W1, Wsrc1, Wdst1], axis=1)
    P1 = _tc_matmul(x, Wcat1)
    h1 = P1[:, :128]
    zpad = jnp.zeros((N, 8), jnp.float32)
    A1 = jnp.concatenate([P1[:, 128:136], zpad], axis=1)
    A2_1 = jnp.concatenate([P1[:, 136:144], zpad], axis=1)

    acc1 = _sc_pass_l1(src, dst, A1, A2_1, h1)[:, :N, :]

    rep = (jax.lax.broadcasted_iota(jnp.int32, (8, 128), 1) // 16
           == jax.lax.broadcasted_iota(jnp.int32, (8, 128), 0)
           ).astype(jnp.float32)
    Wsrc2 = (W2.reshape(128, H2, F2) * att_src2[None]).sum(-1)
    Wdst2 = (W2.reshape(128, H2, F2) * att_dst2[None]).sum(-1)
    Wcat2 = jnp.concatenate(
        [W2, Wsrc2, Wdst2, jnp.zeros((128, 62), jnp.float32)], axis=1)
    P2 = _tc_epilogue1(acc1, b1.reshape(1, 128), rep, Wcat2)

    h2 = P2[:, :64]
    zpad15 = jnp.zeros((N, 15), jnp.float32)
    A1_2 = jnp.concatenate([P2[:, 64:65], zpad15], axis=1)
    A2_2 = jnp.concatenate([P2[:, 65:66], zpad15], axis=1)

    acc2 = _sc_pass_l2(src, dst, A1_2, A2_2, h2)[:, :N, :]

    return _tc_epilogue2(acc2, b2.reshape(1, 64))


# speedup vs baseline: 1.0065x; 1.0065x over previous
import functools

import jax
import jax.numpy as jnp
from jax import lax
from jax.experimental import pallas as pl
from jax.experimental.pallas import tpu as pltpu
from jax.experimental.pallas import tpu_sc as plsc

N = 10000
E = 320000
NC = 2
NS = 16
LANES = 16
NW = NC * NS
EPW = E // NW
CH = 80
NCHUNK = EPW // CH
NP = 10240
RPT = NP // NS
ZR = 8


def _make_sc_edge_pass(H, F, NB):
    HF = H * F
    ROW = HF + 16
    mesh = plsc.VectorSubcoreMesh(core_axis_name="c", subcore_axis_name="s")

    def _pair(ty):
        return [ty] * NB

    @functools.partial(
        pl.kernel,
        out_type=jax.ShapeDtypeStruct((NC, NP, ROW), jnp.float32),
        mesh=mesh,
        compiler_params=pltpu.CompilerParams(
            needs_layout_passes=False, use_tc_tiling_on_sc=False),
        scratch_types=[
            pltpu.VMEM((CH,), jnp.int32),
            _pair(pltpu.VMEM((CH,), jnp.int32)),
            pltpu.VMEM((CH, 16), jnp.float32),
            pltpu.VMEM((CH, 16), jnp.float32),
            pltpu.VMEM((CH, HF), jnp.float32),
            _pair(pltpu.VMEM((CH, ROW), jnp.float32)),
            pltpu.VMEM((CH * LANES,), jnp.float32),
            pltpu.VMEM_SHARED((NP, ROW), jnp.float32),
            pltpu.SemaphoreType.DMA,
            _pair(pltpu.SemaphoreType.DMA),
        ],
    )
    def sc_pass(src_hbm, dst_hbm, a_hbm, a2_hbm, h_hbm, out_hbm,
                srcv, dstv, asrcv, adstv, hv, rowv, wv, accum,
                sem_g, sem_sc):
        cid = lax.axis_index("c")
        sid = lax.axis_index("s")
        wid = cid * NS + sid
        zvec = jnp.zeros((LANES,), jnp.float32)
        lane = lax.iota(jnp.int32, LANES)
        wmask = lane < H

        @pl.loop(0, CH)
        def _(r):
            for k in range(ROW // LANES):
                rowv[0][r, pl.ds(k * LANES, LANES)] = zvec

        @pl.loop(0, RPT // CH)
        def _(z):
            pltpu.sync_copy(rowv[0], accum.at[pl.ds(sid * RPT + z * CH, CH)])

        plsc.subcore_barrier()

        def wait_scatter(p):
            pltpu.make_async_copy(
                rowv[p], accum.at[dstv[p]], sem_sc[p]).wait()

        def chunk_body(p, ci, guarded):
            if guarded:
                @pl.when(ci >= NB)
                def _():
                    wait_scatter(p)
            else:
                wait_scatter(p)
            off = wid * EPW + ci * CH
            pltpu.sync_copy(src_hbm.at[pl.ds(off, CH)], srcv)
            pltpu.sync_copy(dst_hbm.at[pl.ds(off, CH)], dstv[p])
            c1 = pltpu.async_copy(a_hbm.at[srcv], asrcv, sem_g)
            c2 = pltpu.async_copy(a2_hbm.at[dstv[p]], adstv, sem_g)
            c3 = pltpu.async_copy(h_hbm.at[srcv], hv, sem_g)
            c1.wait()
            c2.wait()
            c3.wait()

            @pl.loop(0, CH, unroll=2)
            def _(e):
                s = asrcv[e, :] + adstv[e, :]
                s = jnp.where(s >= 0.0, s, 0.2 * s)
                w = jnp.where(wmask, jnp.exp(s), 0.0)
                rowv[p][e, pl.ds(HF, LANES)] = w
                wv[pl.ds(e * LANES, LANES)] = w
                for hd in range(H):
                    whd = plsc.load_gather(
                        wv, [jnp.full((LANES,), e * LANES + hd, jnp.int32)])
                    for fv in range(F // LANES):
                        col = hd * F + fv * LANES
                        rowv[p][e, pl.ds(col, LANES)] = (
                            hv[e, pl.ds(col, LANES)] * whd)

            pltpu.async_copy(rowv[p], accum.at[dstv[p]], sem_sc[p], add=True)

        @pl.loop(0, NCHUNK - 1, step=NB)
        def _(ci):
            for k in range(NB):
                chunk_body(k, ci + k, True)

        chunk_body(0, NCHUNK - 1, False)
        for k in range(1, NB):
            wait_scatter(k)
        wait_scatter(0)

        plsc.subcore_barrier()
        pltpu.sync_copy(accum.at[pl.ds(sid * RPT, RPT)],
                        out_hbm.at[cid, pl.ds(sid * RPT, RPT)])

    return sc_pass


_sc_pass_l1 = _make_sc_edge_pass(8, 16, 2)
_sc_pass_l2 = _make_sc_edge_pass(1, 64, 4)


def _mm_body(x_ref, w_ref, o_ref):
    o_ref[...] = jnp.dot(x_ref[...], w_ref[...],
                         preferred_element_type=jnp.float32)


def _tc_matmul(x, w, block_rows=2000):
    n, k = x.shape
    _, m = w.shape
    return pl.pallas_call(
        _mm_body,
        grid=(n // block_rows,),
        in_specs=[
            pl.BlockSpec((block_rows, k), lambda i: (i, 0)),
            pl.BlockSpec((k, m), lambda i: (0, 0)),
        ],
        out_specs=pl.BlockSpec((block_rows, m), lambda i: (i, 0)),
        out_shape=jax.ShapeDtypeStruct((n, m), jnp.float32),
    )(x, w)


def _ep1_body(acc_ref, b_ref, rep_ref, w2_ref, o_ref):
    a = acc_ref[0] + acc_ref[1]
    msg = a[:, 0:128]
    den = jnp.dot(a[:, 128:136], rep_ref[...],
                  preferred_element_type=jnp.float32)
    out1 = msg / (den + 1e-16) + b_ref[...]
    x2 = jnp.where(out1 > 0.0, out1, jnp.exp(jnp.minimum(out1, 0.0)) - 1.0)
    o_ref[...] = jnp.dot(x2, w2_ref[...], preferred_element_type=jnp.float32)


def _tc_epilogue1(acc, b1, rep, Wcat2, block_rows=2000):
    return pl.pallas_call(
        _ep1_body,
        grid=(N // block_rows,),
        in_specs=[
            pl.BlockSpec((2, block_rows, 144), lambda i: (0, i, 0)),
            pl.BlockSpec((1, 128), lambda i: (0, 0)),
            pl.BlockSpec((8, 128), lambda i: (0, 0)),
            pl.BlockSpec((128, 128), lambda i: (0, 0)),
        ],
        out_specs=pl.BlockSpec((block_rows, 128), lambda i: (i, 0)),
        out_shape=jax.ShapeDtypeStruct((N, 128), jnp.float32),
    )(acc, b1, rep, Wcat2)


def _ep2_body(acc_ref, b_ref, o_ref):
    a = acc_ref[0] + acc_ref[1]
    den = a[:, 64:65]
    o_ref[...] = a[:, 0:64] / (den + 1e-16) + b_ref[...]


def _tc_epilogue2(acc, b2, block_rows=2000):
    return pl.pallas_call(
        _ep2_body,
        grid=(N // block_rows,),
        in_specs=[
            pl.BlockSpec((2, block_rows, 80), lambda i: (0, i, 0)),
            pl.BlockSpec((1, 64), lambda i: (0, 0)),
        ],
        out_specs=pl.BlockSpec((block_rows, 64), lambda i: (i, 0)),
        out_shape=jax.ShapeDtypeStruct((N, 64), jnp.float32),
    )(acc, b2)


def kernel(x, edge_index, W1, att_src1, att_dst1, b1, W2, att_src2, att_dst2, b2):
    src = edge_index[0].astype(jnp.int32)
    dst = edge_index[1].astype(jnp.int32)

    H1, F1 = att_src1.shape
    H2, F2 = att_src2.shape

    Wsrc1 = (W1.reshape(128, H1, F1) * att_src1[None]).sum(-1)
    Wdst1 = (W1.reshape(128, H1, F1) * att_dst1[None]).sum(-1)
    Wcat1 = jnp.concatenate([W1, Wsrc1, Wdst1], axis=1)
    P1 = _tc_matmul(x, Wcat1)
    h1 = P1[:, :128]
    zpad = jnp.zeros((N, 8), jnp.float32)
    A1 = jnp.concatenate([P1[:, 128:136], zpad], axis=1)
    A2_1 = jnp.concatenate([P1[:, 136:144], zpad], axis=1)

    acc1 = _sc_pass_l1(src, dst, A1, A2_1, h1)[:, :N, :]

    rep = (jax.lax.broadcasted_iota(jnp.int32, (8, 128), 1) // 16
           == jax.lax.broadcasted_iota(jnp.int32, (8, 128), 0)
           ).astype(jnp.float32)
    Wsrc2 = (W2.reshape(128, H2, F2) * att_src2[None]).sum(-1)
    Wdst2 = (W2.reshape(128, H2, F2) * att_dst2[None]).sum(-1)
    Wcat2 = jnp.concatenate(
        [W2, Wsrc2, Wdst2, jnp.zeros((128, 62), jnp.float32)], axis=1)
    P2 = _tc_epilogue1(acc1, b1.reshape(1, 128), rep, Wcat2)

    h2 = P2[:, :64]
    zpad15 = jnp.zeros((N, 15), jnp.float32)
    A1_2 = jnp.concatenate([P2[:, 64:65], zpad15], axis=1)
    A2_2 = jnp.concatenate([P2[:, 65:66], zpad15], axis=1)

    acc2 = _sc_pass_l2(src, dst, A1_2, A2_2, h2)[:, :N, :]

    return _tc_epilogue2(acc2, b2.reshape(1, 64))


# speedup vs baseline: 1.0069x; 1.0004x over previous
import functools

import jax
import jax.numpy as jnp
from jax import lax
from jax.experimental import pallas as pl
from jax.experimental.pallas import tpu as pltpu
from jax.experimental.pallas import tpu_sc as plsc

N = 10000
E = 320000
NC = 2
NS = 16
LANES = 16
NW = NC * NS
EPW = E // NW
CH = 80
NCHUNK = EPW // CH
NP = 10240
RPT = NP // NS


def _make_sc_edge_pass(H, F, NB):
    HF = H * F
    ROW = HF + 16
    mesh = plsc.VectorSubcoreMesh(core_axis_name="c", subcore_axis_name="s")

    def _pair(ty):
        return [ty] * NB

    @functools.partial(
        pl.kernel,
        out_type=jax.ShapeDtypeStruct((NC, NP, ROW), jnp.float32),
        mesh=mesh,
        compiler_params=pltpu.CompilerParams(
            needs_layout_passes=False, use_tc_tiling_on_sc=False),
        scratch_types=[
            pltpu.VMEM((CH,), jnp.int32),
            _pair(pltpu.VMEM((CH,), jnp.int32)),
            pltpu.VMEM((CH, 16), jnp.float32),
            pltpu.VMEM((CH, 16), jnp.float32),
            pltpu.VMEM((CH, HF), jnp.float32),
            _pair(pltpu.VMEM((CH, ROW), jnp.float32)),
            pltpu.VMEM((CH * LANES,), jnp.float32),
            pltpu.VMEM_SHARED((NP, ROW), jnp.float32),
            pltpu.SemaphoreType.DMA,
            _pair(pltpu.SemaphoreType.DMA),
        ],
    )
    def sc_pass(src_hbm, dst_hbm, a_hbm, a2_hbm, h_hbm, out_hbm,
                srcv, dstv, asrcv, adstv, hv, rowv, wv, accum,
                sem_g, sem_sc):
        cid = lax.axis_index("c")
        sid = lax.axis_index("s")
        wid = cid * NS + sid
        zvec = jnp.zeros((LANES,), jnp.float32)
        lane = lax.iota(jnp.int32, LANES)
        wmask = lane < H

        @pl.loop(0, CH)
        def _(r):
            for k in range(ROW // LANES):
                rowv[0][r, pl.ds(k * LANES, LANES)] = zvec

        @pl.loop(0, RPT // CH)
        def _(z):
            pltpu.sync_copy(rowv[0], accum.at[pl.ds(sid * RPT + z * CH, CH)])

        plsc.subcore_barrier()

        def wait_scatter(p):
            pltpu.make_async_copy(
                rowv[p], accum.at[dstv[p]], sem_sc[p]).wait()

        def chunk_body(p, ci, guarded):
            if guarded:
                @pl.when(ci >= NB)
                def _():
                    wait_scatter(p)
            else:
                wait_scatter(p)
            off = wid * EPW + ci * CH
            pltpu.sync_copy(src_hbm.at[pl.ds(off, CH)], srcv)
            pltpu.sync_copy(dst_hbm.at[pl.ds(off, CH)], dstv[p])
            c1 = pltpu.async_copy(a_hbm.at[srcv], asrcv, sem_g)
            c2 = pltpu.async_copy(a2_hbm.at[dstv[p]], adstv, sem_g)
            c3 = pltpu.async_copy(h_hbm.at[srcv], hv, sem_g)
            c1.wait()
            c2.wait()
            c3.wait()

            @pl.loop(0, CH, unroll=2)
            def _(e):
                s = asrcv[e, :] + adstv[e, :]
                s = jnp.where(s >= 0.0, s, 0.2 * s)
                w = jnp.where(wmask, jnp.exp(s), 0.0)
                rowv[p][e, pl.ds(HF, LANES)] = w
                wv[pl.ds(e * LANES, LANES)] = w
                for hd in range(H):
                    whd = plsc.load_gather(
                        wv, [jnp.full((LANES,), e * LANES + hd, jnp.int32)])
                    for fv in range(F // LANES):
                        col = hd * F + fv * LANES
                        rowv[p][e, pl.ds(col, LANES)] = (
                            hv[e, pl.ds(col, LANES)] * whd)

            pltpu.async_copy(rowv[p], accum.at[dstv[p]], sem_sc[p], add=True)

        @pl.loop(0, NCHUNK - 1, step=NB)
        def _(ci):
            for k in range(NB):
                chunk_body(k, ci + k, True)

        chunk_body(0, NCHUNK - 1, False)
        for k in range(1, NB):
            wait_scatter(k)
        wait_scatter(0)

        plsc.subcore_barrier()
        pltpu.sync_copy(accum.at[pl.ds(sid * RPT, RPT)],
                        out_hbm.at[cid, pl.ds(sid * RPT, RPT)])

    return sc_pass


_sc_pass_l1 = _make_sc_edge_pass(8, 16, 2)
_sc_pass_l2 = _make_sc_edge_pass(1, 64, 4)


def _mm_body(x_ref, w_ref, o_ref):
    o_ref[...] = jnp.dot(x_ref[...], w_ref[...],
                         preferred_element_type=jnp.float32)


def _tc_matmul(x, w, block_rows=2000):
    n, k = x.shape
    _, m = w.shape
    return pl.pallas_call(
        _mm_body,
        grid=(n // block_rows,),
        in_specs=[
            pl.BlockSpec((block_rows, k), lambda i: (i, 0)),
            pl.BlockSpec((k, m), lambda i: (0, 0)),
        ],
        out_specs=pl.BlockSpec((block_rows, m), lambda i: (i, 0)),
        out_shape=jax.ShapeDtypeStruct((n, m), jnp.float32),
    )(x, w)


def _ep1_body(acc_ref, b_ref, rep_ref, w2_ref, o_ref):
    a = acc_ref[0] + acc_ref[1]
    msg = a[:, 0:128]
    den = jnp.dot(a[:, 128:136], rep_ref[...],
                  preferred_element_type=jnp.float32)
    out1 = msg / (den + 1e-16) + b_ref[...]
    x2 = jnp.where(out1 > 0.0, out1, jnp.exp(jnp.minimum(out1, 0.0)) - 1.0)
    o_ref[...] = jnp.dot(x2, w2_ref[...], preferred_element_type=jnp.float32)


def _tc_epilogue1(acc, b1, rep, Wcat2, block_rows=2000):
    return pl.pallas_call(
        _ep1_body,
        grid=(N // block_rows,),
        in_specs=[
            pl.BlockSpec((2, block_rows, 144), lambda i: (0, i, 0)),
            pl.BlockSpec((1, 128), lambda i: (0, 0)),
            pl.BlockSpec((8, 128), lambda i: (0, 0)),
            pl.BlockSpec((128, 128), lambda i: (0, 0)),
        ],
        out_specs=pl.BlockSpec((block_rows, 128), lambda i: (i, 0)),
        out_shape=jax.ShapeDtypeStruct((N, 128), jnp.float32),
    )(acc, b1, rep, Wcat2)


def _ep2_body(acc_ref, b_ref, o_ref):
    a = acc_ref[0] + acc_ref[1]
    den = a[:, 64:65]
    o_ref[...] = a[:, 0:64] / (den + 1e-16) + b_ref[...]


def _tc_epilogue2(acc, b2, block_rows=2000):
    return pl.pallas_call(
        _ep2_body,
        grid=(N // block_rows,),
        in_specs=[
            pl.BlockSpec((2, block_rows, 80), lambda i: (0, i, 0)),
            pl.BlockSpec((1, 64), lambda i: (0, 0)),
        ],
        out_specs=pl.BlockSpec((block_rows, 64), lambda i: (i, 0)),
        out_shape=jax.ShapeDtypeStruct((N, 64), jnp.float32),
    )(acc, b2)


def kernel(x, edge_index, W1, att_src1, att_dst1, b1, W2, att_src2, att_dst2, b2):
    src = edge_index[0].astype(jnp.int32)
    dst = edge_index[1].astype(jnp.int32)

    H1, F1 = att_src1.shape
    H2, F2 = att_src2.shape

    Wsrc1 = (W1.reshape(128, H1, F1) * att_src1[None]).sum(-1)
    Wdst1 = (W1.reshape(128, H1, F1) * att_dst1[None]).sum(-1)
    Wcat1 = jnp.concatenate([W1, Wsrc1, Wdst1], axis=1)
    P1 = _tc_matmul(x, Wcat1)
    h1 = P1[:, :128]
    zpad = jnp.zeros((N, 8), jnp.float32)
    A1 = jnp.concatenate([P1[:, 128:136], zpad], axis=1)
    A2_1 = jnp.concatenate([P1[:, 136:144], zpad], axis=1)

    acc1 = _sc_pass_l1(src, dst, A1, A2_1, h1)[:, :N, :]

    rep = (jax.lax.broadcasted_iota(jnp.int32, (8, 128), 1) // 16
           == jax.lax.broadcasted_iota(jnp.int32, (8, 128), 0)
           ).astype(jnp.float32)
    Wsrc2 = (W2.reshape(128, H2, F2) * att_src2[None]).sum(-1)
    Wdst2 = (W2.reshape(128, H2, F2) * att_dst2[None]).sum(-1)
    Wcat2 = jnp.concatenate(
        [W2, Wsrc2, Wdst2, jnp.zeros((128, 62), jnp.float32)], axis=1)
    P2 = _tc_epilogue1(acc1, b1.reshape(1, 128), rep, Wcat2)

    h2 = P2[:, :64]
    zpad15 = jnp.zeros((N, 15), jnp.float32)
    A1_2 = jnp.concatenate([P2[:, 64:65], zpad15], axis=1)
    A2_2 = jnp.concatenate([P2[:, 65:66], zpad15], axis=1)

    acc2 = _sc_pass_l2(src, dst, A1_2, A2_2, h2)[:, :N, :]

    return _tc_epilogue2(acc2, b2.reshape(1, 64))
